# baseline (speedup 1.0000x reference)
import functools

import jax
import jax.numpy as jnp
from jax import lax
from jax.experimental import pallas as pl
from jax.experimental.pallas import tpu as pltpu
from jax.experimental.pallas import tpu_sc as plsc

_BROW = 512
_USTEP = 10


def _prep_kernel(x_ref, gbias_ref, xt_ref, h0_ref):
    i = pl.program_id(0)
    x = x_ref[...]
    lblk, d = x.shape[1], x.shape[2]
    padrows = _BROW - x.shape[0]
    m = jnp.transpose(jnp.maximum(x + gbias_ref[0], 0.0), (1, 0, 2))
    xt_ref[...] = jnp.concatenate(
        [m, jnp.zeros((lblk, padrows, d), jnp.float32)], axis=1)
    blockmax = jnp.concatenate(
        [jnp.max(x, axis=1), jnp.zeros((padrows, d), jnp.float32)], axis=0)

    @pl.when(i == 0)
    def _():
        h0_ref[...] = blockmax

    @pl.when(i > 0)
    def _():
        h0_ref[...] = jnp.maximum(h0_ref[...], blockmax)


def _sigmoid(x):
    return 0.5 * (jnp.tanh(0.5 * x) + 1.0)


def _gru_kernel(xf_ref, xb_ref, h0_ref,
                wif_ref, whf_ref, bif_ref, bhf_ref,
                wib_ref, whb_ref, bib_ref, bhb_ref,
                outf_ref, outb_ref, hf_s, hb_s):
    l = pl.program_id(0)

    @pl.when(l == 0)
    def _():
        hf_s[...] = h0_ref[...]
        hb_s[...] = h0_ref[...]

    def cell(x, h, wi_ref, wh_ref, bi_ref, bh_ref):
        gi = lax.dot_general(x, wi_ref[...], (((1,), (1,)), ((), ())),
                             preferred_element_type=jnp.float32) + bi_ref[...]
        gh = lax.dot_general(h, wh_ref[...], (((1,), (1,)), ((), ())),
                             preferred_element_type=jnp.float32) + bh_ref[...]
        d = x.shape[1]
        r = _sigmoid(gi[:, :d] + gh[:, :d])
        z = _sigmoid(gi[:, d:2 * d] + gh[:, d:2 * d])
        n = jnp.tanh(gi[:, 2 * d:] + r * gh[:, 2 * d:])
        return (1.0 - z) * n + z * h

    ustep = outf_ref.shape[0]
    hf = hf_s[...]
    hb = hb_s[...]
    for j in range(ustep):
        hf = cell(xf_ref[j], hf, wif_ref, whf_ref, bif_ref, bhf_ref)
        outf_ref[j] = hf
        hb = cell(xb_ref[ustep - 1 - j], hb, wib_ref, whb_ref, bib_ref,
                  bhb_ref)
        outb_ref[ustep - 1 - j] = hb
    hf_s[...] = hf
    hb_s[...] = hb


def _score_kernel(gf_ref, gb_ref, tr_ref, w3_ref, b3_ref,
                  w1_ref, b1_ref, w2_ref, b2_ref, o_ref):
    d = gf_ref.shape[1]
    w3 = w3_ref[...]
    e = (lax.dot_general(gf_ref[...], w3[:, :d], (((1,), (1,)), ((), ())),
                         preferred_element_type=jnp.float32)
         + lax.dot_general(gb_ref[...], w3[:, d:], (((1,), (1,)), ((), ())),
                           preferred_element_type=jnp.float32)
         + b3_ref[...])
    e = jnp.maximum(e, 0.0)
    half = e.shape[0] // 2
    feat = e[:half] + tr_ref[...] - e[half:]
    u = lax.dot_general(w2_ref[...], w1_ref[...], (((1,), (0,)), ((), ())),
                        preferred_element_type=jnp.float32)
    c = jnp.sum(b1_ref[...] * w2_ref[...]) + b2_ref[0, 0]
    o_ref[...] = jnp.sum(feat * u, axis=1, keepdims=True) + c


@functools.lru_cache(maxsize=None)
def _make_gather(nq, d):
    info = plsc.get_sparse_core_info()
    nc, ns = info.num_cores, info.num_subcores
    nw = nc * ns
    per = nq // nw
    mesh = plsc.VectorSubcoreMesh(core_axis_name="c", subcore_axis_name="s")

    @functools.partial(
        pl.kernel, mesh=mesh,
        out_type=[jax.ShapeDtypeStruct((nq, d), jnp.float32),
                  jax.ShapeDtypeStruct((nq, d), jnp.float32)],
        scratch_types=[pltpu.VMEM((per,), jnp.int32),
                       pltpu.VMEM((per, d), jnp.float32),
                       pltpu.VMEM((per, d), jnp.float32),
                       pltpu.SemaphoreType.DMA,
                       pltpu.SemaphoreType.DMA],
    )
    def gather_k(tf_hbm, tb_hbm, ids_hbm, gf_hbm, gb_hbm,
                 idx_v, rf_v, rb_v, sem_f, sem_b):
        wid = lax.axis_index("s") * nc + lax.axis_index("c")
        base = wid * per
        pltpu.sync_copy(ids_hbm.at[pl.ds(base, per)], idx_v)
        cf = pltpu.async_copy(tf_hbm.at[idx_v], rf_v, sem_f)
        cb = pltpu.async_copy(tb_hbm.at[idx_v], rb_v, sem_b)
        cf.wait()
        cb.wait()
        pltpu.sync_copy(rf_v, gf_hbm.at[pl.ds(base, per)])
        pltpu.sync_copy(rb_v, gb_hbm.at[pl.ds(base, per)])

    return gather_k


def kernel(node, target_rel, path_agg, head_ids, tail_ids, gru_bias,
           W_ih_f, W_hh_f, b_ih_f, b_hh_f,
           W_ih_b, W_hh_b, b_ih_b, b_hh_b,
           W3, b3, W1, b1, W2, b2):
    n, d = node.shape
    bq = target_rel.shape[0]
    seq = n // bq
    d3 = 3 * d
    brow = _BROW
    lblk = 8

    node3 = node.reshape(bq, seq, d)

    xt, h0 = pl.pallas_call(
        _prep_kernel,
        grid=(seq // lblk,),
        in_specs=[
            pl.BlockSpec((bq, lblk, d), lambda i: (0, i, 0)),
            pl.BlockSpec((1, d), lambda i: (0, 0)),
        ],
        out_specs=[
            pl.BlockSpec((lblk, brow, d), lambda i: (i, 0, 0)),
            pl.BlockSpec((brow, d), lambda i: (0, 0)),
        ],
        out_shape=[jax.ShapeDtypeStruct((seq, brow, d), jnp.float32),
                   jax.ShapeDtypeStruct((brow, d), jnp.float32)],
        compiler_params=pltpu.CompilerParams(
            dimension_semantics=("arbitrary",)),
    )(node3, gru_bias.reshape(1, d))

    def full(shape):
        return pl.BlockSpec(shape, lambda l: tuple(0 for _ in shape))

    out_f3, out_b3 = pl.pallas_call(
        _gru_kernel,
        grid=(seq // _USTEP,),
        in_specs=[
            pl.BlockSpec((_USTEP, brow, d), lambda g: (g, 0, 0)),
            pl.BlockSpec((_USTEP, brow, d), lambda g: (seq // _USTEP - 1 - g, 0, 0)),
            full((brow, d)),
            full((d3, d)), full((d3, d)), full((1, d3)), full((1, d3)),
            full((d3, d)), full((d3, d)), full((1, d3)), full((1, d3)),
        ],
        out_specs=[
            pl.BlockSpec((_USTEP, brow, d), lambda g: (g, 0, 0)),
            pl.BlockSpec((_USTEP, brow, d), lambda g: (seq // _USTEP - 1 - g, 0, 0)),
        ],
        out_shape=[jax.ShapeDtypeStruct((seq, brow, d), jnp.float32)] * 2,
        scratch_shapes=[pltpu.VMEM((brow, d), jnp.float32)] * 2,
        compiler_params=pltpu.CompilerParams(
            dimension_semantics=("arbitrary",)),
    )(xt, xt, h0,
      W_ih_f, W_hh_f, b_ih_f.reshape(1, d3), b_hh_f.reshape(1, d3),
      W_ih_b, W_hh_b, b_ih_b.reshape(1, d3), b_hh_b.reshape(1, d3))

    tf = out_f3.reshape(seq * brow, d)
    tb = out_b3.reshape(seq * brow, d)

    nq = 1024
    half = nq // 2
    pad = jnp.zeros((half - bq,), jnp.int32)
    ids = jnp.concatenate([head_ids.astype(jnp.int32), pad,
                           tail_ids.astype(jnp.int32), pad])
    ids = jnp.remainder(ids, seq) * brow + ids // seq
    gf, gb = _make_gather(nq, d)(tf, tb, ids)

    tr = jnp.zeros((half, d), jnp.float32).at[:bq].set(target_rel)
    scores = pl.pallas_call(
        _score_kernel,
        out_shape=jax.ShapeDtypeStruct((half, 1), jnp.float32),
    )(gf, gb, tr, W3, b3.reshape(1, d),
      W1, b1.reshape(1, d), W2.reshape(1, d), b2.reshape(1, 1))
    return scores[:bq]


# speedup vs baseline: 1.0018x; 1.0018x over previous
import functools

import jax
import jax.numpy as jnp
from jax import lax
from jax.experimental import pallas as pl
from jax.experimental.pallas import tpu as pltpu
from jax.experimental.pallas import tpu_sc as plsc

_BROW = 512
_USTEP = 20


def _prep_kernel(x_ref, gbias_ref, xt_ref, h0_ref):
    i = pl.program_id(0)
    x = x_ref[...]
    lblk, d = x.shape[1], x.shape[2]
    padrows = _BROW - x.shape[0]
    m = jnp.transpose(jnp.maximum(x + gbias_ref[0], 0.0), (1, 0, 2))
    xt_ref[...] = jnp.concatenate(
        [m, jnp.zeros((lblk, padrows, d), jnp.float32)], axis=1)
    blockmax = jnp.concatenate(
        [jnp.max(x, axis=1), jnp.zeros((padrows, d), jnp.float32)], axis=0)

    @pl.when(i == 0)
    def _():
        h0_ref[...] = blockmax

    @pl.when(i > 0)
    def _():
        h0_ref[...] = jnp.maximum(h0_ref[...], blockmax)


def _sigmoid(x):
    return 0.5 * (jnp.tanh(0.5 * x) + 1.0)


def _gru_kernel(xf_ref, xb_ref, h0_ref,
                wif_ref, whf_ref, bif_ref, bhf_ref,
                wib_ref, whb_ref, bib_ref, bhb_ref,
                outf_ref, outb_ref, hf_s, hb_s):
    l = pl.program_id(0)

    @pl.when(l == 0)
    def _():
        hf_s[...] = h0_ref[...]
        hb_s[...] = h0_ref[...]

    def cell(x, h, wi_ref, wh_ref, bi_ref, bh_ref):
        gi = lax.dot_general(x, wi_ref[...], (((1,), (1,)), ((), ())),
                             preferred_element_type=jnp.float32) + bi_ref[...]
        gh = lax.dot_general(h, wh_ref[...], (((1,), (1,)), ((), ())),
                             preferred_element_type=jnp.float32) + bh_ref[...]
        d = x.shape[1]
        r = _sigmoid(gi[:, :d] + gh[:, :d])
        z = _sigmoid(gi[:, d:2 * d] + gh[:, d:2 * d])
        n = jnp.tanh(gi[:, 2 * d:] + r * gh[:, 2 * d:])
        return (1.0 - z) * n + z * h

    ustep = outf_ref.shape[0]
    hf = hf_s[...]
    hb = hb_s[...]
    for j in range(ustep):
        hf = cell(xf_ref[j], hf, wif_ref, whf_ref, bif_ref, bhf_ref)
        outf_ref[j] = hf
        hb = cell(xb_ref[ustep - 1 - j], hb, wib_ref, whb_ref, bib_ref,
                  bhb_ref)
        outb_ref[ustep - 1 - j] = hb
    hf_s[...] = hf
    hb_s[...] = hb


def _score_kernel(gf_ref, gb_ref, tr_ref, w3_ref, b3_ref,
                  w1_ref, b1_ref, w2_ref, b2_ref, o_ref):
    d = gf_ref.shape[1]
    w3 = w3_ref[...]
    e = (lax.dot_general(gf_ref[...], w3[:, :d], (((1,), (1,)), ((), ())),
                         preferred_element_type=jnp.float32)
         + lax.dot_general(gb_ref[...], w3[:, d:], (((1,), (1,)), ((), ())),
                           preferred_element_type=jnp.float32)
         + b3_ref[...])
    e = jnp.maximum(e, 0.0)
    half = e.shape[0] // 2
    feat = e[:half] + tr_ref[...] - e[half:]
    u = lax.dot_general(w2_ref[...], w1_ref[...], (((1,), (0,)), ((), ())),
                        preferred_element_type=jnp.float32)
    c = jnp.sum(b1_ref[...] * w2_ref[...]) + b2_ref[0, 0]
    o_ref[...] = jnp.sum(feat * u, axis=1, keepdims=True) + c


@functools.lru_cache(maxsize=None)
def _make_gather(nq, d):
    info = plsc.get_sparse_core_info()
    nc, ns = info.num_cores, info.num_subcores
    nw = nc * ns
    per = nq // nw
    mesh = plsc.VectorSubcoreMesh(core_axis_name="c", subcore_axis_name="s")

    @functools.partial(
        pl.kernel, mesh=mesh,
        out_type=[jax.ShapeDtypeStruct((nq, d), jnp.float32),
                  jax.ShapeDtypeStruct((nq, d), jnp.float32)],
        scratch_types=[pltpu.VMEM((per,), jnp.int32),
                       pltpu.VMEM((per, d), jnp.float32),
                       pltpu.VMEM((per, d), jnp.float32),
                       pltpu.SemaphoreType.DMA,
                       pltpu.SemaphoreType.DMA],
    )
    def gather_k(tf_hbm, tb_hbm, ids_hbm, gf_hbm, gb_hbm,
                 idx_v, rf_v, rb_v, sem_f, sem_b):
        wid = lax.axis_index("s") * nc + lax.axis_index("c")
        base = wid * per
        pltpu.sync_copy(ids_hbm.at[pl.ds(base, per)], idx_v)
        cf = pltpu.async_copy(tf_hbm.at[idx_v], rf_v, sem_f)
        cb = pltpu.async_copy(tb_hbm.at[idx_v], rb_v, sem_b)
        cf.wait()
        cb.wait()
        pltpu.sync_copy(rf_v, gf_hbm.at[pl.ds(base, per)])
        pltpu.sync_copy(rb_v, gb_hbm.at[pl.ds(base, per)])

    return gather_k


def kernel(node, target_rel, path_agg, head_ids, tail_ids, gru_bias,
           W_ih_f, W_hh_f, b_ih_f, b_hh_f,
           W_ih_b, W_hh_b, b_ih_b, b_hh_b,
           W3, b3, W1, b1, W2, b2):
    n, d = node.shape
    bq = target_rel.shape[0]
    seq = n // bq
    d3 = 3 * d
    brow = _BROW
    lblk = 8

    node3 = node.reshape(bq, seq, d)

    xt, h0 = pl.pallas_call(
        _prep_kernel,
        grid=(seq // lblk,),
        in_specs=[
            pl.BlockSpec((bq, lblk, d), lambda i: (0, i, 0)),
            pl.BlockSpec((1, d), lambda i: (0, 0)),
        ],
        out_specs=[
            pl.BlockSpec((lblk, brow, d), lambda i: (i, 0, 0)),
            pl.BlockSpec((brow, d), lambda i: (0, 0)),
        ],
        out_shape=[jax.ShapeDtypeStruct((seq, brow, d), jnp.float32),
                   jax.ShapeDtypeStruct((brow, d), jnp.float32)],
        compiler_params=pltpu.CompilerParams(
            dimension_semantics=("arbitrary",)),
    )(node3, gru_bias.reshape(1, d))

    def full(shape):
        return pl.BlockSpec(shape, lambda l: tuple(0 for _ in shape))

    out_f3, out_b3 = pl.pallas_call(
        _gru_kernel,
        grid=(seq // _USTEP,),
        in_specs=[
            pl.BlockSpec((_USTEP, brow, d), lambda g: (g, 0, 0)),
            pl.BlockSpec((_USTEP, brow, d), lambda g: (seq // _USTEP - 1 - g, 0, 0)),
            full((brow, d)),
            full((d3, d)), full((d3, d)), full((1, d3)), full((1, d3)),
            full((d3, d)), full((d3, d)), full((1, d3)), full((1, d3)),
        ],
        out_specs=[
            pl.BlockSpec((_USTEP, brow, d), lambda g: (g, 0, 0)),
            pl.BlockSpec((_USTEP, brow, d), lambda g: (seq // _USTEP - 1 - g, 0, 0)),
        ],
        out_shape=[jax.ShapeDtypeStruct((seq, brow, d), jnp.float32)] * 2,
        scratch_shapes=[pltpu.VMEM((brow, d), jnp.float32)] * 2,
        compiler_params=pltpu.CompilerParams(
            dimension_semantics=("arbitrary",)),
    )(xt, xt, h0,
      W_ih_f, W_hh_f, b_ih_f.reshape(1, d3), b_hh_f.reshape(1, d3),
      W_ih_b, W_hh_b, b_ih_b.reshape(1, d3), b_hh_b.reshape(1, d3))

    tf = out_f3.reshape(seq * brow, d)
    tb = out_b3.reshape(seq * brow, d)

    nq = 1024
    half = nq // 2
    pad = jnp.zeros((half - bq,), jnp.int32)
    ids = jnp.concatenate([head_ids.astype(jnp.int32), pad,
                           tail_ids.astype(jnp.int32), pad])
    ids = jnp.remainder(ids, seq) * brow + ids // seq
    gf, gb = _make_gather(nq, d)(tf, tb, ids)

    tr = jnp.zeros((half, d), jnp.float32).at[:bq].set(target_rel)
    scores = pl.pallas_call(
        _score_kernel,
        out_shape=jax.ShapeDtypeStruct((half, 1), jnp.float32),
    )(gf, gb, tr, W3, b3.reshape(1, d),
      W1, b1.reshape(1, d), W2.reshape(1, d), b2.reshape(1, 1))
    return scores[:bq]


# speedup vs baseline: 1.0043x; 1.0025x over previous
import functools

import jax
import jax.numpy as jnp
from jax import lax
from jax.experimental import pallas as pl
from jax.experimental.pallas import tpu as pltpu
from jax.experimental.pallas import tpu_sc as plsc

_BROW = 512
_USTEP = 20


def _prep_kernel(x_ref, gbias_ref, xt_ref, h0_ref):
    i = pl.program_id(0)
    x = x_ref[...]
    lblk, d = x.shape[1], x.shape[2]
    padrows = _BROW - x.shape[0]
    m = jnp.transpose(jnp.maximum(x + gbias_ref[0], 0.0), (1, 0, 2))
    xt_ref[...] = jnp.concatenate(
        [m, jnp.zeros((lblk, padrows, d), jnp.float32)],
        axis=1).astype(jnp.bfloat16)
    blockmax = jnp.concatenate(
        [jnp.max(x, axis=1), jnp.zeros((padrows, d), jnp.float32)], axis=0)

    @pl.when(i == 0)
    def _():
        h0_ref[...] = blockmax

    @pl.when(i > 0)
    def _():
        h0_ref[...] = jnp.maximum(h0_ref[...], blockmax)


def _sigmoid(x):
    return 0.5 * (jnp.tanh(0.5 * x) + 1.0)


def _gru_kernel(xf_ref, xb_ref, h0_ref,
                wif_ref, whf_ref, bif_ref, bhf_ref,
                wib_ref, whb_ref, bib_ref, bhb_ref,
                outf_ref, outb_ref, hf_s, hb_s):
    l = pl.program_id(0)

    @pl.when(l == 0)
    def _():
        hf_s[...] = h0_ref[...]
        hb_s[...] = h0_ref[...]

    def cell(x, h, wi_ref, wh_ref, bi_ref, bh_ref):
        gi = lax.dot_general(x, wi_ref[...], (((1,), (1,)), ((), ())),
                             preferred_element_type=jnp.float32) + bi_ref[...]
        gh = lax.dot_general(h.astype(jnp.bfloat16), wh_ref[...],
                             (((1,), (1,)), ((), ())),
                             preferred_element_type=jnp.float32) + bh_ref[...]
        d = x.shape[1]
        r = _sigmoid(gi[:, :d] + gh[:, :d])
        z = _sigmoid(gi[:, d:2 * d] + gh[:, d:2 * d])
        n = jnp.tanh(gi[:, 2 * d:] + r * gh[:, 2 * d:])
        return (1.0 - z) * n + z * h

    ustep = outf_ref.shape[0]
    hf = hf_s[...]
    hb = hb_s[...]
    for j in range(ustep):
        hf = cell(xf_ref[j], hf, wif_ref, whf_ref, bif_ref, bhf_ref)
        outf_ref[j] = hf
        hb = cell(xb_ref[ustep - 1 - j], hb, wib_ref, whb_ref, bib_ref,
                  bhb_ref)
        outb_ref[ustep - 1 - j] = hb
    hf_s[...] = hf
    hb_s[...] = hb


def _score_kernel(gf_ref, gb_ref, tr_ref, w3_ref, b3_ref,
                  w1_ref, b1_ref, w2_ref, b2_ref, o_ref):
    d = gf_ref.shape[1]
    w3 = w3_ref[...]
    gf = gf_ref[...].astype(jnp.float32)
    gb = gb_ref[...].astype(jnp.float32)
    e = (lax.dot_general(gf, w3[:, :d], (((1,), (1,)), ((), ())),
                         preferred_element_type=jnp.float32)
         + lax.dot_general(gb, w3[:, d:], (((1,), (1,)), ((), ())),
                           preferred_element_type=jnp.float32)
         + b3_ref[...])
    e = jnp.maximum(e, 0.0)
    half = e.shape[0] // 2
    feat = e[:half] + tr_ref[...] - e[half:]
    u = lax.dot_general(w2_ref[...], w1_ref[...], (((1,), (0,)), ((), ())),
                        preferred_element_type=jnp.float32)
    c = jnp.sum(b1_ref[...] * w2_ref[...]) + b2_ref[0, 0]
    o_ref[...] = jnp.sum(feat * u, axis=1, keepdims=True) + c


@functools.lru_cache(maxsize=None)
def _make_gather(nq, d):
    info = plsc.get_sparse_core_info()
    nc, ns = info.num_cores, info.num_subcores
    nw = nc * ns
    per = nq // nw
    mesh = plsc.VectorSubcoreMesh(core_axis_name="c", subcore_axis_name="s")

    @functools.partial(
        pl.kernel, mesh=mesh,
        out_type=[jax.ShapeDtypeStruct((nq, d), jnp.float32),
                  jax.ShapeDtypeStruct((nq, d), jnp.float32)],
        scratch_types=[pltpu.VMEM((per,), jnp.int32),
                       pltpu.VMEM((per, d), jnp.float32),
                       pltpu.VMEM((per, d), jnp.float32),
                       pltpu.SemaphoreType.DMA,
                       pltpu.SemaphoreType.DMA],
    )
    def gather_k(tf_hbm, tb_hbm, ids_hbm, gf_hbm, gb_hbm,
                 idx_v, rf_v, rb_v, sem_f, sem_b):
        wid = lax.axis_index("s") * nc + lax.axis_index("c")
        base = wid * per
        pltpu.sync_copy(ids_hbm.at[pl.ds(base, per)], idx_v)
        cf = pltpu.async_copy(tf_hbm.at[idx_v], rf_v, sem_f)
        cb = pltpu.async_copy(tb_hbm.at[idx_v], rb_v, sem_b)
        cf.wait()
        cb.wait()
        pltpu.sync_copy(rf_v, gf_hbm.at[pl.ds(base, per)])
        pltpu.sync_copy(rb_v, gb_hbm.at[pl.ds(base, per)])

    return gather_k


def kernel(node, target_rel, path_agg, head_ids, tail_ids, gru_bias,
           W_ih_f, W_hh_f, b_ih_f, b_hh_f,
           W_ih_b, W_hh_b, b_ih_b, b_hh_b,
           W3, b3, W1, b1, W2, b2):
    n, d = node.shape
    bq = target_rel.shape[0]
    seq = n // bq
    d3 = 3 * d
    brow = _BROW
    lblk = 8

    node3 = node.reshape(bq, seq, d)

    xt, h0 = pl.pallas_call(
        _prep_kernel,
        grid=(seq // lblk,),
        in_specs=[
            pl.BlockSpec((bq, lblk, d), lambda i: (0, i, 0)),
            pl.BlockSpec((1, d), lambda i: (0, 0)),
        ],
        out_specs=[
            pl.BlockSpec((lblk, brow, d), lambda i: (i, 0, 0)),
            pl.BlockSpec((brow, d), lambda i: (0, 0)),
        ],
        out_shape=[jax.ShapeDtypeStruct((seq, brow, d), jnp.bfloat16),
                   jax.ShapeDtypeStruct((brow, d), jnp.float32)],
        compiler_params=pltpu.CompilerParams(
            dimension_semantics=("arbitrary",)),
    )(node3, gru_bias.reshape(1, d))

    def full(shape):
        return pl.BlockSpec(shape, lambda l: tuple(0 for _ in shape))

    out_f3, out_b3 = pl.pallas_call(
        _gru_kernel,
        grid=(seq // _USTEP,),
        in_specs=[
            pl.BlockSpec((_USTEP, brow, d), lambda g: (g, 0, 0)),
            pl.BlockSpec((_USTEP, brow, d), lambda g: (seq // _USTEP - 1 - g, 0, 0)),
            full((brow, d)),
            full((d3, d)), full((d3, d)), full((1, d3)), full((1, d3)),
            full((d3, d)), full((d3, d)), full((1, d3)), full((1, d3)),
        ],
        out_specs=[
            pl.BlockSpec((_USTEP, brow, d), lambda g: (g, 0, 0)),
            pl.BlockSpec((_USTEP, brow, d), lambda g: (seq // _USTEP - 1 - g, 0, 0)),
        ],
        out_shape=[jax.ShapeDtypeStruct((seq, brow, d), jnp.float32)] * 2,
        scratch_shapes=[pltpu.VMEM((brow, d), jnp.float32)] * 2,
        compiler_params=pltpu.CompilerParams(
            dimension_semantics=("arbitrary",)),
    )(xt, xt, h0,
      W_ih_f.astype(jnp.bfloat16), W_hh_f.astype(jnp.bfloat16),
      b_ih_f.reshape(1, d3), b_hh_f.reshape(1, d3),
      W_ih_b.astype(jnp.bfloat16), W_hh_b.astype(jnp.bfloat16),
      b_ih_b.reshape(1, d3), b_hh_b.reshape(1, d3))

    tf = out_f3.reshape(seq * brow, d)
    tb = out_b3.reshape(seq * brow, d)

    nq = 1024
    half = nq // 2
    pad = jnp.zeros((half - bq,), jnp.int32)
    ids = jnp.concatenate([head_ids.astype(jnp.int32), pad,
                           tail_ids.astype(jnp.int32), pad])
    ids = jnp.remainder(ids, seq) * brow + ids // seq
    gf, gb = _make_gather(nq, d)(tf, tb, ids)

    tr = jnp.zeros((half, d), jnp.float32).at[:bq].set(target_rel)
    scores = pl.pallas_call(
        _score_kernel,
        out_shape=jax.ShapeDtypeStruct((half, 1), jnp.float32),
    )(gf, gb, tr, W3, b3.reshape(1, d),
      W1, b1.reshape(1, d), W2.reshape(1, d), b2.reshape(1, 1))
    return scores[:bq]


# speedup vs baseline: 1.0728x; 1.0682x over previous
import functools

import jax
import jax.numpy as jnp
from jax import lax
from jax.experimental import pallas as pl
from jax.experimental.pallas import tpu as pltpu
from jax.experimental.pallas import tpu_sc as plsc

_BROW = 512
_USTEP = 25


def _prep_kernel(x_ref, gbias_ref, xt_ref, h0_ref):
    i = pl.program_id(0)
    x = x_ref[...]
    lblk, d = x.shape[1], x.shape[2]
    padrows = _BROW - x.shape[0]
    m = jnp.transpose(jnp.maximum(x + gbias_ref[0], 0.0), (1, 0, 2))
    xt_ref[...] = jnp.concatenate(
        [m, jnp.zeros((lblk, padrows, d), jnp.float32)],
        axis=1).astype(jnp.bfloat16)
    blockmax = jnp.concatenate(
        [jnp.max(x, axis=1), jnp.zeros((padrows, d), jnp.float32)], axis=0)

    @pl.when(i == 0)
    def _():
        h0_ref[...] = blockmax

    @pl.when(i > 0)
    def _():
        h0_ref[...] = jnp.maximum(h0_ref[...], blockmax)


def _sigmoid(x):
    return 0.5 * (jnp.tanh(0.5 * x) + 1.0)


def _gru_kernel(xf_ref, xb_ref, h0_ref,
                wif_ref, whf_ref, bif_ref, bhf_ref,
                wib_ref, whb_ref, bib_ref, bhb_ref,
                outf_ref, outb_ref, hf_s, hb_s):
    l = pl.program_id(0)

    @pl.when(l == 0)
    def _():
        hf_s[...] = h0_ref[...]
        hb_s[...] = h0_ref[...]

    def cell(x, h, wi_ref, wh_ref, bi_ref, bh_ref):
        gi = lax.dot_general(x, wi_ref[...], (((1,), (1,)), ((), ())),
                             preferred_element_type=jnp.float32) + bi_ref[...]
        gh = lax.dot_general(h.astype(jnp.bfloat16), wh_ref[...],
                             (((1,), (1,)), ((), ())),
                             preferred_element_type=jnp.float32) + bh_ref[...]
        d = x.shape[1]
        r = _sigmoid(gi[:, :d] + gh[:, :d])
        z = _sigmoid(gi[:, d:2 * d] + gh[:, d:2 * d])
        n = jnp.tanh(gi[:, 2 * d:] + r * gh[:, 2 * d:])
        return (1.0 - z) * n + z * h

    ustep = outf_ref.shape[0]
    hf = hf_s[...]
    hb = hb_s[...]
    for j in range(ustep):
        hf = cell(xf_ref[j], hf, wif_ref, whf_ref, bif_ref, bhf_ref)
        outf_ref[j] = hf
        hb = cell(xb_ref[ustep - 1 - j], hb, wib_ref, whb_ref, bib_ref,
                  bhb_ref)
        outb_ref[ustep - 1 - j] = hb
    hf_s[...] = hf
    hb_s[...] = hb


def _score_kernel(gf_ref, gb_ref, tr_ref, w3_ref, b3_ref,
                  w1_ref, b1_ref, w2_ref, b2_ref, o_ref):
    d = gf_ref.shape[1]
    w3 = w3_ref[...]
    gf = gf_ref[...].astype(jnp.float32)
    gb = gb_ref[...].astype(jnp.float32)
    e = (lax.dot_general(gf, w3[:, :d], (((1,), (1,)), ((), ())),
                         preferred_element_type=jnp.float32)
         + lax.dot_general(gb, w3[:, d:], (((1,), (1,)), ((), ())),
                           preferred_element_type=jnp.float32)
         + b3_ref[...])
    e = jnp.maximum(e, 0.0)
    half = e.shape[0] // 2
    feat = e[:half] + tr_ref[...] - e[half:]
    u = lax.dot_general(w2_ref[...], w1_ref[...], (((1,), (0,)), ((), ())),
                        preferred_element_type=jnp.float32)
    c = jnp.sum(b1_ref[...] * w2_ref[...]) + b2_ref[0, 0]
    o_ref[...] = jnp.sum(feat * u, axis=1, keepdims=True) + c


@functools.lru_cache(maxsize=None)
def _make_gather(nq, d):
    info = plsc.get_sparse_core_info()
    nc, ns = info.num_cores, info.num_subcores
    nw = nc * ns
    per = nq // nw
    mesh = plsc.VectorSubcoreMesh(core_axis_name="c", subcore_axis_name="s")

    @functools.partial(
        pl.kernel, mesh=mesh,
        out_type=[jax.ShapeDtypeStruct((nq, d), jnp.float32),
                  jax.ShapeDtypeStruct((nq, d), jnp.float32)],
        scratch_types=[pltpu.VMEM((per,), jnp.int32),
                       pltpu.VMEM((per, d), jnp.float32),
                       pltpu.VMEM((per, d), jnp.float32),
                       pltpu.SemaphoreType.DMA,
                       pltpu.SemaphoreType.DMA],
    )
    def gather_k(tf_hbm, tb_hbm, ids_hbm, gf_hbm, gb_hbm,
                 idx_v, rf_v, rb_v, sem_f, sem_b):
        wid = lax.axis_index("s") * nc + lax.axis_index("c")
        base = wid * per
        pltpu.sync_copy(ids_hbm.at[pl.ds(base, per)], idx_v)
        cf = pltpu.async_copy(tf_hbm.at[idx_v], rf_v, sem_f)
        cb = pltpu.async_copy(tb_hbm.at[idx_v], rb_v, sem_b)
        cf.wait()
        cb.wait()
        pltpu.sync_copy(rf_v, gf_hbm.at[pl.ds(base, per)])
        pltpu.sync_copy(rb_v, gb_hbm.at[pl.ds(base, per)])

    return gather_k


def kernel(node, target_rel, path_agg, head_ids, tail_ids, gru_bias,
           W_ih_f, W_hh_f, b_ih_f, b_hh_f,
           W_ih_b, W_hh_b, b_ih_b, b_hh_b,
           W3, b3, W1, b1, W2, b2):
    n, d = node.shape
    bq = target_rel.shape[0]
    seq = n // bq
    d3 = 3 * d
    brow = _BROW
    lblk = 40

    node3 = node.reshape(bq, seq, d)

    xt, h0 = pl.pallas_call(
        _prep_kernel,
        grid=(seq // lblk,),
        in_specs=[
            pl.BlockSpec((bq, lblk, d), lambda i: (0, i, 0)),
            pl.BlockSpec((1, d), lambda i: (0, 0)),
        ],
        out_specs=[
            pl.BlockSpec((lblk, brow, d), lambda i: (i, 0, 0)),
            pl.BlockSpec((brow, d), lambda i: (0, 0)),
        ],
        out_shape=[jax.ShapeDtypeStruct((seq, brow, d), jnp.bfloat16),
                   jax.ShapeDtypeStruct((brow, d), jnp.float32)],
        compiler_params=pltpu.CompilerParams(
            dimension_semantics=("arbitrary",)),
    )(node3, gru_bias.reshape(1, d))

    def full(shape):
        return pl.BlockSpec(shape, lambda l: tuple(0 for _ in shape))

    out_f3, out_b3 = pl.pallas_call(
        _gru_kernel,
        grid=(seq // _USTEP,),
        in_specs=[
            pl.BlockSpec((_USTEP, brow, d), lambda g: (g, 0, 0)),
            pl.BlockSpec((_USTEP, brow, d), lambda g: (seq // _USTEP - 1 - g, 0, 0)),
            full((brow, d)),
            full((d3, d)), full((d3, d)), full((1, d3)), full((1, d3)),
            full((d3, d)), full((d3, d)), full((1, d3)), full((1, d3)),
        ],
        out_specs=[
            pl.BlockSpec((_USTEP, brow, d), lambda g: (g, 0, 0)),
            pl.BlockSpec((_USTEP, brow, d), lambda g: (seq // _USTEP - 1 - g, 0, 0)),
        ],
        out_shape=[jax.ShapeDtypeStruct((seq, brow, d), jnp.float32)] * 2,
        scratch_shapes=[pltpu.VMEM((brow, d), jnp.float32)] * 2,
        compiler_params=pltpu.CompilerParams(
            dimension_semantics=("arbitrary",)),
    )(xt, xt, h0,
      W_ih_f.astype(jnp.bfloat16), W_hh_f.astype(jnp.bfloat16),
      b_ih_f.reshape(1, d3), b_hh_f.reshape(1, d3),
      W_ih_b.astype(jnp.bfloat16), W_hh_b.astype(jnp.bfloat16),
      b_ih_b.reshape(1, d3), b_hh_b.reshape(1, d3))

    tf = out_f3.reshape(seq * brow, d)
    tb = out_b3.reshape(seq * brow, d)

    nq = 1024
    half = nq // 2
    pad = jnp.zeros((half - bq,), jnp.int32)
    ids = jnp.concatenate([head_ids.astype(jnp.int32), pad,
                           tail_ids.astype(jnp.int32), pad])
    ids = jnp.remainder(ids, seq) * brow + ids // seq
    gf, gb = _make_gather(nq, d)(tf, tb, ids)

    tr = jnp.zeros((half, d), jnp.float32).at[:bq].set(target_rel)
    scores = pl.pallas_call(
        _score_kernel,
        out_shape=jax.ShapeDtypeStruct((half, 1), jnp.float32),
    )(gf, gb, tr, W3, b3.reshape(1, d),
      W1, b1.reshape(1, d), W2.reshape(1, d), b2.reshape(1, 1))
    return scores[:bq]


# speedup vs baseline: 1.0835x; 1.0100x over previous
import functools

import jax
import jax.numpy as jnp
from jax import lax
from jax.experimental import pallas as pl
from jax.experimental.pallas import tpu as pltpu
from jax.experimental.pallas import tpu_sc as plsc

_BROW = 512
_USTEP = 25


def _prep_kernel(x_ref, gbias_ref, xt_ref, h0_ref):
    i = pl.program_id(0)
    x = x_ref[...]
    lblk, d = x.shape[1], x.shape[2]
    padrows = _BROW - x.shape[0]
    m = jnp.transpose(jnp.maximum(x + gbias_ref[0], 0.0), (1, 0, 2))
    xt_ref[...] = jnp.concatenate(
        [m, jnp.zeros((lblk, padrows, d), jnp.float32)],
        axis=1).astype(jnp.bfloat16)
    blockmax = jnp.concatenate(
        [jnp.max(x, axis=1), jnp.zeros((padrows, d), jnp.float32)], axis=0)

    @pl.when(i == 0)
    def _():
        h0_ref[...] = blockmax

    @pl.when(i > 0)
    def _():
        h0_ref[...] = jnp.maximum(h0_ref[...], blockmax)


def _sigmoid(x):
    return 0.5 * (jnp.tanh(0.5 * x) + 1.0)


def _gru_kernel(xf_ref, xb_ref, h0_ref,
                wif_ref, whf_ref, wib_ref, whb_ref, bias_ref,
                outf_ref, outb_ref, hf_s, hb_s):
    l = pl.program_id(0)
    bias = bias_ref[...]
    bif, bhf = bias[0:1], bias[1:2]
    bib, bhb = bias[2:3], bias[3:4]

    @pl.when(l == 0)
    def _():
        hf_s[...] = h0_ref[...]
        hb_s[...] = h0_ref[...]

    def cell(x, h, wi_ref, wh_ref, bi, bh):
        gi = lax.dot_general(x, wi_ref[...], (((1,), (1,)), ((), ())),
                             preferred_element_type=jnp.float32) + bi
        gh = lax.dot_general(h.astype(jnp.bfloat16), wh_ref[...],
                             (((1,), (1,)), ((), ())),
                             preferred_element_type=jnp.float32) + bh
        d = x.shape[1]
        r = _sigmoid(gi[:, :d] + gh[:, :d])
        z = _sigmoid(gi[:, d:2 * d] + gh[:, d:2 * d])
        n = jnp.tanh(gi[:, 2 * d:] + r * gh[:, 2 * d:])
        return (1.0 - z) * n + z * h

    ustep = outf_ref.shape[0]
    hf = hf_s[...]
    hb = hb_s[...]
    for j in range(ustep):
        hf = cell(xf_ref[j], hf, wif_ref, whf_ref, bif, bhf)
        outf_ref[j] = hf
        hb = cell(xb_ref[ustep - 1 - j], hb, wib_ref, whb_ref, bib, bhb)
        outb_ref[ustep - 1 - j] = hb
    hf_s[...] = hf
    hb_s[...] = hb


def _score_kernel(gf_ref, gb_ref, tr_ref, w3_ref, b3_ref,
                  w1_ref, b1_ref, w2_ref, b2_ref, o_ref):
    d = gf_ref.shape[1]
    w3 = w3_ref[...]
    gf = gf_ref[...].astype(jnp.float32)
    gb = gb_ref[...].astype(jnp.float32)
    e = (lax.dot_general(gf, w3[:, :d], (((1,), (1,)), ((), ())),
                         preferred_element_type=jnp.float32)
         + lax.dot_general(gb, w3[:, d:], (((1,), (1,)), ((), ())),
                           preferred_element_type=jnp.float32)
         + b3_ref[...])
    e = jnp.maximum(e, 0.0)
    half = e.shape[0] // 2
    feat = e[:half] + tr_ref[...] - e[half:]
    u = lax.dot_general(w2_ref[...], w1_ref[...], (((1,), (0,)), ((), ())),
                        preferred_element_type=jnp.float32)
    c = jnp.sum(b1_ref[...] * w2_ref[...]) + b2_ref[0, 0]
    o_ref[...] = jnp.sum(feat * u, axis=1, keepdims=True) + c


@functools.lru_cache(maxsize=None)
def _make_gather(nq, d):
    info = plsc.get_sparse_core_info()
    nc, ns = info.num_cores, info.num_subcores
    nw = nc * ns
    per = nq // nw
    mesh = plsc.VectorSubcoreMesh(core_axis_name="c", subcore_axis_name="s")

    @functools.partial(
        pl.kernel, mesh=mesh,
        out_type=[jax.ShapeDtypeStruct((nq, d), jnp.float32),
                  jax.ShapeDtypeStruct((nq, d), jnp.float32)],
        scratch_types=[pltpu.VMEM((per,), jnp.int32),
                       pltpu.VMEM((per, d), jnp.float32),
                       pltpu.VMEM((per, d), jnp.float32),
                       pltpu.SemaphoreType.DMA,
                       pltpu.SemaphoreType.DMA],
    )
    def gather_k(tf_hbm, tb_hbm, ids_hbm, gf_hbm, gb_hbm,
                 idx_v, rf_v, rb_v, sem_f, sem_b):
        wid = lax.axis_index("s") * nc + lax.axis_index("c")
        base = wid * per
        pltpu.sync_copy(ids_hbm.at[pl.ds(base, per)], idx_v)
        cf = pltpu.async_copy(tf_hbm.at[idx_v], rf_v, sem_f)
        cb = pltpu.async_copy(tb_hbm.at[idx_v], rb_v, sem_b)
        cf.wait()
        cb.wait()
        pltpu.sync_copy(rf_v, gf_hbm.at[pl.ds(base, per)])
        pltpu.sync_copy(rb_v, gb_hbm.at[pl.ds(base, per)])

    return gather_k


def kernel(node, target_rel, path_agg, head_ids, tail_ids, gru_bias,
           W_ih_f, W_hh_f, b_ih_f, b_hh_f,
           W_ih_b, W_hh_b, b_ih_b, b_hh_b,
           W3, b3, W1, b1, W2, b2):
    n, d = node.shape
    bq = target_rel.shape[0]
    seq = n // bq
    d3 = 3 * d
    brow = _BROW
    lblk = 40

    node3 = node.reshape(bq, seq, d)

    xt, h0 = pl.pallas_call(
        _prep_kernel,
        grid=(seq // lblk,),
        in_specs=[
            pl.BlockSpec((bq, lblk, d), lambda i: (0, i, 0)),
            pl.BlockSpec((1, d), lambda i: (0, 0)),
        ],
        out_specs=[
            pl.BlockSpec((lblk, brow, d), lambda i: (i, 0, 0)),
            pl.BlockSpec((brow, d), lambda i: (0, 0)),
        ],
        out_shape=[jax.ShapeDtypeStruct((seq, brow, d), jnp.bfloat16),
                   jax.ShapeDtypeStruct((brow, d), jnp.float32)],
        compiler_params=pltpu.CompilerParams(
            dimension_semantics=("arbitrary",)),
    )(node3, gru_bias.reshape(1, d))

    def full(shape):
        return pl.BlockSpec(shape, lambda l: tuple(0 for _ in shape))

    out_f3, out_b3 = pl.pallas_call(
        _gru_kernel,
        grid=(seq // _USTEP,),
        in_specs=[
            pl.BlockSpec((_USTEP, brow, d), lambda g: (g, 0, 0)),
            pl.BlockSpec((_USTEP, brow, d), lambda g: (seq // _USTEP - 1 - g, 0, 0)),
            full((brow, d)),
            full((d3, d)), full((d3, d)), full((d3, d)), full((d3, d)),
            full((4, d3)),
        ],
        out_specs=[
            pl.BlockSpec((_USTEP, brow, d), lambda g: (g, 0, 0)),
            pl.BlockSpec((_USTEP, brow, d), lambda g: (seq // _USTEP - 1 - g, 0, 0)),
        ],
        out_shape=[jax.ShapeDtypeStruct((seq, brow, d), jnp.float32)] * 2,
        scratch_shapes=[pltpu.VMEM((brow, d), jnp.float32)] * 2,
        compiler_params=pltpu.CompilerParams(
            dimension_semantics=("arbitrary",)),
    )(xt, xt, h0,
      W_ih_f.astype(jnp.bfloat16), W_hh_f.astype(jnp.bfloat16),
      W_ih_b.astype(jnp.bfloat16), W_hh_b.astype(jnp.bfloat16),
      jnp.stack([b_ih_f, b_hh_f, b_ih_b, b_hh_b]))

    tf = out_f3.reshape(seq * brow, d)
    tb = out_b3.reshape(seq * brow, d)

    nq = 1024
    half = nq // 2
    pad = jnp.zeros((half - bq,), jnp.int32)
    ids = jnp.concatenate([head_ids.astype(jnp.int32), pad,
                           tail_ids.astype(jnp.int32), pad])
    ids = jnp.remainder(ids, seq) * brow + ids // seq
    gf, gb = _make_gather(nq, d)(tf, tb, ids)

    tr = jnp.zeros((half, d), jnp.float32).at[:bq].set(target_rel)
    scores = pl.pallas_call(
        _score_kernel,
        out_shape=jax.ShapeDtypeStruct((half, 1), jnp.float32),
    )(gf, gb, tr, W3, b3.reshape(1, d),
      W1, b1.reshape(1, d), W2.reshape(1, d), b2.reshape(1, 1))
    return scores[:bq]
